# baseline (speedup 1.0000x reference)
import functools

import jax
import jax.numpy as jnp
from jax import lax
from jax.experimental import pallas as pl
from jax.experimental.pallas import tpu as pltpu
from jax.experimental.pallas import tpu_sc as plsc

NUM_CORES = 2
NUM_SUBCORES = 16
NW = NUM_CORES * NUM_SUBCORES
BATCH = 16384
SEQ = 100
EMB = 32
BPW = BATCH // NW
GRP = 4
PAIR = GRP * SEQ
PPW = BPW // GRP
NB = 4
ROW_UNROLL = 4
OUT_ROWS = BPW * EMB // 128


def _sc_body(in_hbm, tab_hbm, out_hbm, idx_v, rows_v, out_v, sems):
    wid = lax.axis_index("s") * NUM_CORES + lax.axis_index("c")

    pltpu.sync_copy(in_hbm.at[pl.ds(wid * PPW, PPW), :], idx_v)

    def fire(pr, b):
        pltpu.async_copy(tab_hbm.at[idx_v.at[pr]], rows_v.at[b], sems.at[b])

    def wait(pr, b):
        pltpu.make_async_copy(
            tab_hbm.at[idx_v.at[pr]], rows_v.at[b], sems.at[b]
        ).wait()

    def reduce_rows(rows_ref):
        zero = jnp.zeros((16,), jnp.float32)
        accs = (zero,) * (2 * ROW_UNROLL)

        def body(r, carry):
            acc = list(carry)
            r0 = r * ROW_UNROLL
            for j in range(ROW_UNROLL):
                acc[2 * j] = acc[2 * j] + rows_ref[r0 + j, 0:16]
                acc[2 * j + 1] = acc[2 * j + 1] + rows_ref[r0 + j, 16:32]
            return tuple(acc)

        accs = lax.fori_loop(0, SEQ // ROW_UNROLL, body, accs)
        lo = (accs[0] + accs[2]) + (accs[4] + accs[6])
        hi = (accs[1] + accs[3]) + (accs[5] + accs[7])
        scale = jnp.float32(1.0 / SEQ)
        return lo * scale, hi * scale

    for b in range(NB):
        fire(b, b)

    def outer(g, carry):
        for b in range(NB):
            pr = g * NB + b
            wait(pr, b)

            nxt = pr + NB

            @pl.when(nxt < PPW)
            def _():
                fire(nxt, b)

            for half in range(GRP):
                lo, hi = reduce_rows(rows_v.at[b, pl.ds(half * SEQ, SEQ)])
                e = GRP * pr + half
                r_i = e // 4
                c0 = pl.multiple_of((e % 4) * EMB, 32)
                out_v[r_i, pl.ds(c0, 16)] = lo
                out_v[r_i, pl.ds(c0 + 16, 16)] = hi
        return carry

    lax.fori_loop(0, PPW // NB, outer, 0)

    pltpu.sync_copy(out_v, out_hbm.at[pl.ds(wid * OUT_ROWS, OUT_ROWS), :])


_embed_bag = functools.partial(
    pl.kernel,
    out_type=jax.ShapeDtypeStruct((BATCH * EMB // 128, 128), jnp.float32),
    mesh=plsc.VectorSubcoreMesh(
        core_axis_name="c",
        subcore_axis_name="s",
        num_cores=NUM_CORES,
        num_subcores=NUM_SUBCORES,
    ),
    scratch_types=[
        pltpu.VMEM((PPW, PAIR), jnp.int32),
        pltpu.VMEM((NB, PAIR, EMB), jnp.float32),
        pltpu.VMEM((OUT_ROWS, 128), jnp.float32),
        pltpu.SemaphoreType.DMA((NB,)),
    ],
    compiler_params=pltpu.CompilerParams(use_tc_tiling_on_sc=False),
)(_sc_body)


@jax.jit
def kernel(input, table):
    idx = input.astype(jnp.int32).reshape(BATCH // GRP, PAIR)
    out = _embed_bag(idx, table)
    return out.reshape(BATCH, EMB)
